# Initial kernel scaffold; baseline (speedup 1.0000x reference)
import jax
import jax.numpy as jnp
from jax.experimental import pallas as pl


def kernel(assessment, maximize):
    raise NotImplementedError("write your pallas kernel here")


# speedup vs baseline: 2.3056x; 2.3056x over previous
import jax
import jax.numpy as jnp
from jax import lax
from jax.experimental import pallas as pl
from jax.experimental.pallas import tpu as pltpu
from jax.experimental.pallas import tpu_sc as plsc

POP = 8192
BATCH = 128
NSEL = 2048
NQ = 2 * NSEL
CHUNK = 128
NCHUNK = POP // CHUNK
L = 16
NW = 32
ROWS_PER_W = BATCH // NW


def _rot(v, k, lanes):
    key = (lanes - k) & (L - 1)
    _, out = plsc.sort_key_val(key, v)
    return out


def _allmax(v, lanes):
    for k in (8, 4, 2, 1):
        v = jnp.maximum(v, _rot(v, k, lanes))
    return v


def _sc_body(at_hbm, u_hbm, sgn_hbm, sel_hbm, val_hbm,
             arow, urow, sgnv, et, cumt, cumn, selrow, valrow):
    wid = lax.axis_index("s") * 2 + lax.axis_index("c")
    lanes = lax.iota(jnp.int32, L)
    pltpu.sync_copy(sgn_hbm, sgnv)
    sgn = sgnv[...]

    for rr in range(ROWS_PER_W):
        row = wid * ROWS_PER_W + rr
        pltpu.sync_copy(at_hbm.at[row], arow)
        pltpu.sync_copy(u_hbm.at[row], urow)

        def amax_body(i, mvec):
            a = arow[pl.ds(i * L, L)]
            return jnp.maximum(mvec, a * sgn)
        mvec = lax.fori_loop(0, POP // L, amax_body,
                             jnp.full((L,), -jnp.inf, jnp.float32))
        m_splat = _allmax(mvec, lanes)

        def expz_body(i, zacc):
            a = arow[pl.ds(i * L, L)]
            e = jnp.exp(a * sgn - m_splat)
            taddr = lanes * NCHUNK + ((i & 7) * (L * NCHUNK) + (i >> 3))
            plsc.store_scatter(et, [taddr], e)
            idx2 = (lanes & 7) * NCHUNK + ((i & 7) * (L * NCHUNK) + (i >> 3))
            ga = plsc.load_gather(et, [idx2])
            gb = plsc.load_gather(et, [idx2 + 8 * NCHUNK])
            return (zacc + ga) + gb
        zacc = lax.fori_loop(0, POP // L, expz_body,
                             jnp.zeros((L,), jnp.float32))
        t = zacc + _rot(zacc, 4, lanes)
        t = t + _rot(t, 2, lanes)
        z_splat = t + _rot(t, 1, lanes)
        r_splat = 1.0 / z_splat

        def scan_body(pos, carr):
            base = pos * NCHUNK
            out = []
            for g in range(NCHUNK // L):
                e = et[pl.ds(base + g * L, L)]
                cg = carr[g] + e * r_splat
                cumt[pl.ds(base + g * L, L)] = cg
                out.append(cg)
            return tuple(out)
        z16 = jnp.zeros((L,), jnp.float32)
        lax.fori_loop(0, CHUNK, scan_body, (z16, z16, z16, z16))

        def last_body(c, acc):
            t = plsc.load_gather(
                cumt, [jnp.full((L,), (CHUNK - 1) * NCHUNK + c, jnp.int32)])
            return acc + t
        last_splat = lax.fori_loop(0, NCHUNK, last_body, z16)
        lrcp_splat = 1.0 / last_splat

        def offs_body(c, offs):
            for j in range(CHUNK // L):
                idx = lanes * NCHUNK + (j * (L * NCHUNK) + c)
                g = plsc.load_gather(cumt, [idx])
                cumn[pl.ds(c * CHUNK + j * L, L)] = (g + offs) * lrcp_splat
            t = plsc.load_gather(
                cumt, [jnp.full((L,), (CHUNK - 1) * NCHUNK + c, jnp.int32)])
            return offs + t
        lax.fori_loop(0, NCHUNK, offs_body, z16)

        def search_body(q, carry):
            u = urow[pl.ds(q * L, L)]
            lo = jnp.zeros((L,), jnp.int32)
            s = POP // 2
            while s >= 1:
                g = plsc.load_gather(cumn, [lo + (s - 1)])
                lo = lo + jnp.where(g < u, s, 0)
                s //= 2
            v = plsc.load_gather(arow, [lo])
            selrow[pl.ds(q * L, L)] = lo
            valrow[pl.ds(q * L, L)] = v
            return carry
        lax.fori_loop(0, NQ // L, search_body, 0)

        pltpu.sync_copy(selrow, sel_hbm.at[row])
        pltpu.sync_copy(valrow, val_hbm.at[row])


def _sc_call(at, u2, sgn):
    mesh = plsc.VectorSubcoreMesh(core_axis_name="c", subcore_axis_name="s")
    f = pl.kernel(
        _sc_body,
        mesh=mesh,
        compiler_params=pltpu.CompilerParams(needs_layout_passes=False),
        out_type=[jax.ShapeDtypeStruct((BATCH, NQ), jnp.int32),
                  jax.ShapeDtypeStruct((BATCH, NQ), jnp.float32)],
        scratch_types=[
            pltpu.VMEM((POP,), jnp.float32),
            pltpu.VMEM((NQ,), jnp.float32),
            pltpu.VMEM((L,), jnp.float32),
            pltpu.VMEM((POP,), jnp.float32),
            pltpu.VMEM((POP,), jnp.float32),
            pltpu.VMEM((POP,), jnp.float32),
            pltpu.VMEM((NQ,), jnp.int32),
            pltpu.VMEM((NQ,), jnp.float32),
        ],
    )
    return f(at, u2, sgn)


def kernel(assessment, maximize=False):
    key = jax.random.key(42)
    u = jax.random.uniform(key, (2, BATCH, NSEL))
    u2 = jnp.swapaxes(u, 0, 1).reshape(BATCH, NQ)
    at = jnp.swapaxes(assessment, 0, 1)
    maxi = jnp.asarray(maximize, jnp.bool_)
    sgn = jnp.where(maxi, jnp.float32(1), jnp.float32(-1)) * jnp.ones(
        (L,), jnp.float32)
    sel2, val2 = _sc_call(at, u2, sgn)
    sel = sel2.reshape(BATCH, 2, NSEL)
    val = val2.reshape(BATCH, 2, NSEL)
    return (jnp.swapaxes(val[:, 0, :], 0, 1),
            jnp.swapaxes(sel[:, 0, :], 0, 1),
            jnp.swapaxes(val[:, 1, :], 0, 1),
            jnp.swapaxes(sel[:, 1, :], 0, 1))


# speedup vs baseline: 2.3663x; 1.0263x over previous
import jax
import jax.numpy as jnp
from jax import lax
from jax.experimental import pallas as pl
from jax.experimental.pallas import tpu as pltpu
from jax.experimental.pallas import tpu_sc as plsc

POP = 8192
BATCH = 128
NSEL = 2048
NQ = 2 * NSEL
CHUNK = 128
NCHUNK = POP // CHUNK
L = 16
NW = 32
ROWS_PER_W = BATCH // NW


def _rot(v, k, lanes):
    key = (lanes - k) & (L - 1)
    _, out = plsc.sort_key_val(key, v)
    return out


def _allmax(v, lanes):
    for k in (8, 4, 2, 1):
        v = jnp.maximum(v, _rot(v, k, lanes))
    return v


def _sc_body(at_hbm, u_hbm, sgn_hbm, sel_hbm, val_hbm,
             arow, urow, sgnv, et, cumt, cumn, selrow, valrow):
    wid = lax.axis_index("s") * 2 + lax.axis_index("c")
    lanes = lax.iota(jnp.int32, L)
    pltpu.sync_copy(sgn_hbm, sgnv)
    sgn = sgnv[...]

    for rr in range(ROWS_PER_W):
        row = wid * ROWS_PER_W + rr
        pltpu.sync_copy(at_hbm.at[row], arow)
        pltpu.sync_copy(u_hbm.at[row], urow)

        def amax_body(i, mvec):
            a = arow[pl.ds(i * L, L)]
            return jnp.maximum(mvec, a * sgn)
        mvec = lax.fori_loop(0, POP // L, amax_body,
                             jnp.full((L,), -jnp.inf, jnp.float32),
                             unroll=4)
        m_splat = _allmax(mvec, lanes)

        def expz_body(i, zacc):
            a = arow[pl.ds(i * L, L)]
            e = jnp.exp(a * sgn - m_splat)
            taddr = lanes * NCHUNK + ((i & 7) * (L * NCHUNK) + (i >> 3))
            plsc.store_scatter(et, [taddr], e)
            idx2 = (lanes & 7) * NCHUNK + ((i & 7) * (L * NCHUNK) + (i >> 3))
            ga = plsc.load_gather(et, [idx2])
            gb = plsc.load_gather(et, [idx2 + 8 * NCHUNK])
            return (zacc + ga) + gb
        zacc = lax.fori_loop(0, POP // L, expz_body,
                             jnp.zeros((L,), jnp.float32), unroll=4)
        t = zacc + _rot(zacc, 4, lanes)
        t = t + _rot(t, 2, lanes)
        z_splat = t + _rot(t, 1, lanes)
        r_splat = 1.0 / z_splat

        def scan_body(pos, carr):
            base = pos * NCHUNK
            out = []
            for g in range(NCHUNK // L):
                e = et[pl.ds(base + g * L, L)]
                cg = carr[g] + e * r_splat
                cumt[pl.ds(base + g * L, L)] = cg
                out.append(cg)
            return tuple(out)
        z16 = jnp.zeros((L,), jnp.float32)
        lax.fori_loop(0, CHUNK, scan_body, (z16, z16, z16, z16))

        def last_body(c, acc):
            t = plsc.load_gather(
                cumt, [jnp.full((L,), (CHUNK - 1) * NCHUNK + c, jnp.int32)])
            return acc + t
        last_splat = lax.fori_loop(0, NCHUNK, last_body, z16)
        lrcp_splat = 1.0 / last_splat

        def offs_body(c, offs):
            for j in range(CHUNK // L):
                idx = lanes * NCHUNK + (j * (L * NCHUNK) + c)
                g = plsc.load_gather(cumt, [idx])
                cumn[pl.ds(c * CHUNK + j * L, L)] = (g + offs) * lrcp_splat
            t = plsc.load_gather(
                cumt, [jnp.full((L,), (CHUNK - 1) * NCHUNK + c, jnp.int32)])
            return offs + t
        lax.fori_loop(0, NCHUNK, offs_body, z16)

        def search_body(q, carry):
            u = urow[pl.ds(q * L, L)]
            lo = jnp.zeros((L,), jnp.int32)
            s = POP // 2
            while s >= 1:
                g = plsc.load_gather(cumn, [lo + (s - 1)])
                lo = lo + jnp.where(g < u, s, 0)
                s //= 2
            v = plsc.load_gather(arow, [lo])
            selrow[pl.ds(q * L, L)] = lo
            valrow[pl.ds(q * L, L)] = v
            return carry
        lax.fori_loop(0, NQ // L, search_body, 0, unroll=8)

        pltpu.sync_copy(selrow, sel_hbm.at[row])
        pltpu.sync_copy(valrow, val_hbm.at[row])


def _sc_call(at, u2, sgn):
    mesh = plsc.VectorSubcoreMesh(core_axis_name="c", subcore_axis_name="s")
    f = pl.kernel(
        _sc_body,
        mesh=mesh,
        compiler_params=pltpu.CompilerParams(needs_layout_passes=False),
        out_type=[jax.ShapeDtypeStruct((BATCH, NQ), jnp.int32),
                  jax.ShapeDtypeStruct((BATCH, NQ), jnp.float32)],
        scratch_types=[
            pltpu.VMEM((POP,), jnp.float32),
            pltpu.VMEM((NQ,), jnp.float32),
            pltpu.VMEM((L,), jnp.float32),
            pltpu.VMEM((POP,), jnp.float32),
            pltpu.VMEM((POP,), jnp.float32),
            pltpu.VMEM((POP,), jnp.float32),
            pltpu.VMEM((NQ,), jnp.int32),
            pltpu.VMEM((NQ,), jnp.float32),
        ],
    )
    return f(at, u2, sgn)


def kernel(assessment, maximize=False):
    key = jax.random.key(42)
    u = jax.random.uniform(key, (2, BATCH, NSEL))
    u2 = jnp.swapaxes(u, 0, 1).reshape(BATCH, NQ)
    at = jnp.swapaxes(assessment, 0, 1)
    maxi = jnp.asarray(maximize, jnp.bool_)
    sgn = jnp.where(maxi, jnp.float32(1), jnp.float32(-1)) * jnp.ones(
        (L,), jnp.float32)
    sel2, val2 = _sc_call(at, u2, sgn)
    sel = sel2.reshape(BATCH, 2, NSEL)
    val = val2.reshape(BATCH, 2, NSEL)
    return (jnp.swapaxes(val[:, 0, :], 0, 1),
            jnp.swapaxes(sel[:, 0, :], 0, 1),
            jnp.swapaxes(val[:, 1, :], 0, 1),
            jnp.swapaxes(sel[:, 1, :], 0, 1))


# speedup vs baseline: 3.4113x; 1.4416x over previous
import jax
import jax.numpy as jnp
from jax import lax
from jax.experimental import pallas as pl
from jax.experimental.pallas import tpu as pltpu
from jax.experimental.pallas import tpu_sc as plsc

POP = 8192
BATCH = 128
NSEL = 2048
NQ = 2 * NSEL
CHUNK = 128
NCHUNK = POP // CHUNK
L = 16
NW = 32
ROWS_PER_W = BATCH // NW


def _rot(v, k, lanes):
    key = (lanes - k) & (L - 1)
    _, out = plsc.sort_key_val(key, v)
    return out


def _allmax(v, lanes):
    for k in (8, 4, 2, 1):
        v = jnp.maximum(v, _rot(v, k, lanes))
    return v


def _sc_body(at_hbm, u_hbm, sgn_hbm, sel_hbm, val_hbm,
             arow, urow, sgnv, et, cumt, cumn, selrow, valrow):
    wid = lax.axis_index("s") * 2 + lax.axis_index("c")
    lanes = lax.iota(jnp.int32, L)
    pltpu.sync_copy(sgn_hbm, sgnv)
    sgn = sgnv[...]

    for rr in range(ROWS_PER_W):
        row = wid * ROWS_PER_W + rr
        pltpu.sync_copy(at_hbm.at[row], arow)
        pltpu.sync_copy(u_hbm.at[row], urow)

        def amax_body(i, mvec):
            a = arow[pl.ds(i * L, L)]
            return jnp.maximum(mvec, a * sgn)
        mvec = plsc.parallel_loop(
            0, POP // L, carry=jnp.full((L,), -jnp.inf, jnp.float32),
            unroll=4)(amax_body)
        m_splat = _allmax(mvec, lanes)

        def expz_body(i, zacc):
            a = arow[pl.ds(i * L, L)]
            e = jnp.exp(a * sgn - m_splat)
            taddr = lanes * NCHUNK + ((i & 7) * (L * NCHUNK) + (i >> 3))
            plsc.store_scatter(et, [taddr], e)
            idx2 = (lanes & 7) * NCHUNK + ((i & 7) * (L * NCHUNK) + (i >> 3))
            ga = plsc.load_gather(et, [idx2])
            gb = plsc.load_gather(et, [idx2 + 8 * NCHUNK])
            return (zacc + ga) + gb
        zacc = lax.fori_loop(0, POP // L, expz_body,
                             jnp.zeros((L,), jnp.float32), unroll=4)
        t = zacc + _rot(zacc, 4, lanes)
        t = t + _rot(t, 2, lanes)
        z_splat = t + _rot(t, 1, lanes)
        r_splat = 1.0 / z_splat

        def scan_body(pos, carr):
            base = pos * NCHUNK
            out = []
            for g in range(NCHUNK // L):
                e = et[pl.ds(base + g * L, L)]
                cg = carr[g] + e * r_splat
                cumt[pl.ds(base + g * L, L)] = cg
                out.append(cg)
            return tuple(out)
        z16 = jnp.zeros((L,), jnp.float32)
        plsc.parallel_loop(0, CHUNK, carry=(z16, z16, z16, z16),
                           unroll=2)(scan_body)

        def last_body(c, acc):
            t = plsc.load_gather(
                cumt, [jnp.full((L,), (CHUNK - 1) * NCHUNK + c, jnp.int32)])
            return acc + t
        last_splat = plsc.parallel_loop(0, NCHUNK, carry=z16,
                                        unroll=4)(last_body)
        lrcp_splat = 1.0 / last_splat

        def offs_body(c, offs):
            for j in range(CHUNK // L):
                idx = lanes * NCHUNK + (j * (L * NCHUNK) + c)
                g = plsc.load_gather(cumt, [idx])
                cumn[pl.ds(c * CHUNK + j * L, L)] = (g + offs) * lrcp_splat
            t = plsc.load_gather(
                cumt, [jnp.full((L,), (CHUNK - 1) * NCHUNK + c, jnp.int32)])
            return offs + t
        plsc.parallel_loop(0, NCHUNK, carry=z16, unroll=2)(offs_body)

        def search_body(q):
            u = urow[pl.ds(q * L, L)]
            lo = jnp.zeros((L,), jnp.int32)
            s = POP // 2
            while s >= 1:
                g = plsc.load_gather(cumn, [lo + (s - 1)])
                lo = lo + jnp.where(g < u, s, 0)
                s //= 2
            v = plsc.load_gather(arow, [lo])
            selrow[pl.ds(q * L, L)] = lo
            valrow[pl.ds(q * L, L)] = v
        plsc.parallel_loop(0, NQ // L, unroll=8)(search_body)

        pltpu.sync_copy(selrow, sel_hbm.at[row])
        pltpu.sync_copy(valrow, val_hbm.at[row])


def _sc_call(at, u2, sgn):
    mesh = plsc.VectorSubcoreMesh(core_axis_name="c", subcore_axis_name="s")
    f = pl.kernel(
        _sc_body,
        mesh=mesh,
        compiler_params=pltpu.CompilerParams(needs_layout_passes=False),
        out_type=[jax.ShapeDtypeStruct((BATCH, NQ), jnp.int32),
                  jax.ShapeDtypeStruct((BATCH, NQ), jnp.float32)],
        scratch_types=[
            pltpu.VMEM((POP,), jnp.float32),
            pltpu.VMEM((NQ,), jnp.float32),
            pltpu.VMEM((L,), jnp.float32),
            pltpu.VMEM((POP,), jnp.float32),
            pltpu.VMEM((POP,), jnp.float32),
            pltpu.VMEM((POP,), jnp.float32),
            pltpu.VMEM((NQ,), jnp.int32),
            pltpu.VMEM((NQ,), jnp.float32),
        ],
    )
    return f(at, u2, sgn)


def kernel(assessment, maximize=False):
    key = jax.random.key(42)
    u = jax.random.uniform(key, (2, BATCH, NSEL))
    u2 = jnp.swapaxes(u, 0, 1).reshape(BATCH, NQ)
    at = jnp.swapaxes(assessment, 0, 1)
    maxi = jnp.asarray(maximize, jnp.bool_)
    sgn = jnp.where(maxi, jnp.float32(1), jnp.float32(-1)) * jnp.ones(
        (L,), jnp.float32)
    sel2, val2 = _sc_call(at, u2, sgn)
    sel = sel2.reshape(BATCH, 2, NSEL)
    val = val2.reshape(BATCH, 2, NSEL)
    return (jnp.swapaxes(val[:, 0, :], 0, 1),
            jnp.swapaxes(sel[:, 0, :], 0, 1),
            jnp.swapaxes(val[:, 1, :], 0, 1),
            jnp.swapaxes(sel[:, 1, :], 0, 1))


# speedup vs baseline: 4.5513x; 1.3342x over previous
import jax
import jax.numpy as jnp
from jax import lax
from jax.experimental import pallas as pl
from jax.experimental.pallas import tpu as pltpu
from jax.experimental.pallas import tpu_sc as plsc

POP = 8192
BATCH = 128
NSEL = 2048
NQ = 2 * NSEL
CHUNK = 128
NCHUNK = POP // CHUNK
PADC = NCHUNK + 1
L = 16
NW = 32
ROWS_PER_W = BATCH // NW


def _rot(v, k, lanes):
    key = (lanes - k) & (L - 1)
    _, out = plsc.sort_key_val(key, v)
    return out


def _allmax(v, lanes):
    for k in (8, 4, 2, 1):
        v = jnp.maximum(v, _rot(v, k, lanes))
    return v


def _sc_body(at_hbm, u_hbm, sgn_hbm, sel_hbm, val_hbm,
             arow, urow, sgnv, et, cumt, cumn, selrow, valrow):
    wid = lax.axis_index("s") * 2 + lax.axis_index("c")
    lanes = lax.iota(jnp.int32, L)
    pltpu.sync_copy(sgn_hbm, sgnv)
    sgn = sgnv[...]

    for rr in range(ROWS_PER_W):
        row = wid * ROWS_PER_W + rr
        pltpu.sync_copy(at_hbm.at[row], arow)
        pltpu.sync_copy(u_hbm.at[row], urow)

        def amax_body(i, mvec):
            a = arow[pl.ds(i * L, L)]
            return jnp.maximum(mvec, a * sgn)
        mvec = plsc.parallel_loop(
            0, POP // L, carry=jnp.full((L,), -jnp.inf, jnp.float32),
            unroll=4)(amax_body)
        m_splat = _allmax(mvec, lanes)

        def expz_body(i, zacc):
            a = arow[pl.ds(i * L, L)]
            e = jnp.exp(a * sgn - m_splat)
            taddr = lanes * PADC + ((i & 7) * (L * PADC) + (i >> 3))
            plsc.store_scatter(et, [taddr], e)
            idx2 = (lanes & 7) * PADC + ((i & 7) * (L * PADC) + (i >> 3))
            ga = plsc.load_gather(et, [idx2])
            gb = plsc.load_gather(et, [idx2 + 8 * PADC])
            return (zacc + ga) + gb
        zacc = lax.fori_loop(0, POP // L, expz_body,
                             jnp.zeros((L,), jnp.float32), unroll=4)
        t = zacc + _rot(zacc, 4, lanes)
        t = t + _rot(t, 2, lanes)
        z_splat = t + _rot(t, 1, lanes)
        r_splat = 1.0 / z_splat

        def scan_body(pos, carr):
            base = pos * PADC
            out = []
            for g in range(NCHUNK // L):
                e = et[pl.ds(base + g * L, L)]
                cg = carr[g] + e * r_splat
                cumt[pl.ds(base + g * L, L)] = cg
                out.append(cg)
            return tuple(out)
        z16 = jnp.zeros((L,), jnp.float32)
        plsc.parallel_loop(0, CHUNK, carry=(z16, z16, z16, z16),
                           unroll=2)(scan_body)

        def last_body(c, acc):
            t = plsc.load_gather(
                cumt, [jnp.full((L,), (CHUNK - 1) * PADC + c, jnp.int32)])
            return acc + t
        last_splat = plsc.parallel_loop(0, NCHUNK, carry=z16,
                                        unroll=4)(last_body)
        lrcp_splat = 1.0 / last_splat

        def offs_body(c, offs):
            for j in range(CHUNK // L):
                idx = lanes * PADC + (j * (L * PADC) + c)
                g = plsc.load_gather(cumt, [idx])
                cumn[pl.ds(c * CHUNK + j * L, L)] = (g + offs) * lrcp_splat
            t = plsc.load_gather(
                cumt, [jnp.full((L,), (CHUNK - 1) * PADC + c, jnp.int32)])
            return offs + t
        plsc.parallel_loop(0, NCHUNK, carry=z16, unroll=2)(offs_body)

        def search_body(q):
            u = urow[pl.ds(q * L, L)]
            lo = jnp.zeros((L,), jnp.int32)
            s = POP // 2
            while s >= 1:
                g = plsc.load_gather(cumn, [lo + (s - 1)])
                lo = lo + jnp.where(g < u, s, 0)
                s //= 2
            v = plsc.load_gather(arow, [lo])
            selrow[pl.ds(q * L, L)] = lo
            valrow[pl.ds(q * L, L)] = v
        plsc.parallel_loop(0, NQ // L, unroll=8)(search_body)

        pltpu.sync_copy(selrow, sel_hbm.at[row])
        pltpu.sync_copy(valrow, val_hbm.at[row])


def _sc_call(at, u2, sgn):
    mesh = plsc.VectorSubcoreMesh(core_axis_name="c", subcore_axis_name="s")
    f = pl.kernel(
        _sc_body,
        mesh=mesh,
        compiler_params=pltpu.CompilerParams(needs_layout_passes=False),
        out_type=[jax.ShapeDtypeStruct((BATCH, NQ), jnp.int32),
                  jax.ShapeDtypeStruct((BATCH, NQ), jnp.float32)],
        scratch_types=[
            pltpu.VMEM((POP,), jnp.float32),
            pltpu.VMEM((NQ,), jnp.float32),
            pltpu.VMEM((L,), jnp.float32),
            pltpu.VMEM((CHUNK * PADC,), jnp.float32),
            pltpu.VMEM((CHUNK * PADC,), jnp.float32),
            pltpu.VMEM((POP,), jnp.float32),
            pltpu.VMEM((NQ,), jnp.int32),
            pltpu.VMEM((NQ,), jnp.float32),
        ],
    )
    return f(at, u2, sgn)


def kernel(assessment, maximize=False):
    key = jax.random.key(42)
    u = jax.random.uniform(key, (2, BATCH, NSEL))
    u2 = jnp.swapaxes(u, 0, 1).reshape(BATCH, NQ)
    at = jnp.swapaxes(assessment, 0, 1)
    maxi = jnp.asarray(maximize, jnp.bool_)
    sgn = jnp.where(maxi, jnp.float32(1), jnp.float32(-1)) * jnp.ones(
        (L,), jnp.float32)
    sel2, val2 = _sc_call(at, u2, sgn)
    sel = sel2.reshape(BATCH, 2, NSEL)
    val = val2.reshape(BATCH, 2, NSEL)
    return (jnp.swapaxes(val[:, 0, :], 0, 1),
            jnp.swapaxes(sel[:, 0, :], 0, 1),
            jnp.swapaxes(val[:, 1, :], 0, 1),
            jnp.swapaxes(sel[:, 1, :], 0, 1))


# speedup vs baseline: 4.7124x; 1.0354x over previous
import jax
import jax.numpy as jnp
from jax import lax
from jax.experimental import pallas as pl
from jax.experimental.pallas import tpu as pltpu
from jax.experimental.pallas import tpu_sc as plsc

POP = 8192
BATCH = 128
NSEL = 2048
NQ = 2 * NSEL
CHUNK = 128
NCHUNK = POP // CHUNK
PADC = NCHUNK + 1
L = 16
NW = 32
ROWS_PER_W = BATCH // NW


def _rot(v, k, lanes):
    key = (lanes - k) & (L - 1)
    _, out = plsc.sort_key_val(key, v)
    return out


def _allmax(v, lanes):
    for k in (8, 4, 2, 1):
        v = jnp.maximum(v, _rot(v, k, lanes))
    return v


def _sc_body(at_hbm, u_hbm, sgn_hbm, sel_hbm, val_hbm,
             arow, urow, sgnv, et, cumt, cumn, selrow, valrow):
    wid = lax.axis_index("s") * 2 + lax.axis_index("c")
    lanes = lax.iota(jnp.int32, L)
    pltpu.sync_copy(sgn_hbm, sgnv)
    sgn = sgnv[...]

    for rr in range(ROWS_PER_W):
        row = wid * ROWS_PER_W + rr
        pltpu.sync_copy(at_hbm.at[row], arow)
        pltpu.sync_copy(u_hbm.at[row], urow)

        def amax_body(i, mvec):
            a = arow[pl.ds(i * L, L)]
            return jnp.maximum(mvec, a * sgn)
        mvec = plsc.parallel_loop(
            0, POP // L, carry=jnp.full((L,), -jnp.inf, jnp.float32),
            unroll=4)(amax_body)
        m_splat = _allmax(mvec, lanes)

        lo8 = lanes < 8
        def expz_body(i, zacc):
            a = arow[pl.ds(i * L, L)]
            e = jnp.exp(a * sgn - m_splat)
            taddr = lanes * PADC + ((i & 7) * (L * PADC) + (i >> 3))
            plsc.store_scatter(et, [taddr], e)
            h2 = _rot(e, 8, lanes)
            ga = jnp.where(lo8, e, h2)
            gb = jnp.where(lo8, h2, e)
            return (zacc + ga) + gb
        zacc = lax.fori_loop(0, POP // L, expz_body,
                             jnp.zeros((L,), jnp.float32), unroll=4)
        t = zacc + _rot(zacc, 4, lanes)
        t = t + _rot(t, 2, lanes)
        z_splat = t + _rot(t, 1, lanes)
        r_splat = 1.0 / z_splat

        def scan_body(pos, carr):
            base = pos * PADC
            out = []
            for g in range(NCHUNK // L):
                e = et[pl.ds(base + g * L, L)]
                cg = carr[g] + e * r_splat
                cumt[pl.ds(base + g * L, L)] = cg
                out.append(cg)
            return tuple(out)
        z16 = jnp.zeros((L,), jnp.float32)
        plsc.parallel_loop(0, CHUNK, carry=(z16, z16, z16, z16),
                           unroll=2)(scan_body)

        def last_body(c, acc):
            t = plsc.load_gather(
                cumt, [jnp.full((L,), (CHUNK - 1) * PADC + c, jnp.int32)])
            return acc + t
        last_splat = plsc.parallel_loop(0, NCHUNK, carry=z16,
                                        unroll=4)(last_body)
        lrcp_splat = 1.0 / last_splat

        def offs_body(c, offs):
            for j in range(CHUNK // L):
                idx = lanes * PADC + (j * (L * PADC) + c)
                g = plsc.load_gather(cumt, [idx])
                cumn[pl.ds(c * CHUNK + j * L, L)] = (g + offs) * lrcp_splat
            t = plsc.load_gather(
                cumt, [jnp.full((L,), (CHUNK - 1) * PADC + c, jnp.int32)])
            return offs + t
        plsc.parallel_loop(0, NCHUNK, carry=z16, unroll=2)(offs_body)

        c_mid = plsc.load_gather(cumn, [jnp.full((L,), POP // 2 - 1, jnp.int32)])
        c_q1 = plsc.load_gather(cumn, [jnp.full((L,), POP // 4 - 1, jnp.int32)])
        c_q3 = plsc.load_gather(cumn, [jnp.full((L,), 3 * POP // 4 - 1, jnp.int32)])
        def search_body(q):
            u = urow[pl.ds(q * L, L)]
            m1 = c_mid < u
            lo = jnp.where(m1, POP // 2, 0)
            g2 = jnp.where(m1, c_q3, c_q1)
            lo = lo + jnp.where(g2 < u, POP // 4, 0)
            s = POP // 8
            while s >= 1:
                g = plsc.load_gather(cumn, [lo + (s - 1)])
                lo = lo + jnp.where(g < u, s, 0)
                s //= 2
            v = plsc.load_gather(arow, [lo])
            selrow[pl.ds(q * L, L)] = lo
            valrow[pl.ds(q * L, L)] = v
        plsc.parallel_loop(0, NQ // L, unroll=8)(search_body)

        pltpu.sync_copy(selrow, sel_hbm.at[row])
        pltpu.sync_copy(valrow, val_hbm.at[row])


def _sc_call(at, u2, sgn):
    mesh = plsc.VectorSubcoreMesh(core_axis_name="c", subcore_axis_name="s")
    f = pl.kernel(
        _sc_body,
        mesh=mesh,
        compiler_params=pltpu.CompilerParams(needs_layout_passes=False),
        out_type=[jax.ShapeDtypeStruct((BATCH, NQ), jnp.int32),
                  jax.ShapeDtypeStruct((BATCH, NQ), jnp.float32)],
        scratch_types=[
            pltpu.VMEM((POP,), jnp.float32),
            pltpu.VMEM((NQ,), jnp.float32),
            pltpu.VMEM((L,), jnp.float32),
            pltpu.VMEM((CHUNK * PADC,), jnp.float32),
            pltpu.VMEM((CHUNK * PADC,), jnp.float32),
            pltpu.VMEM((POP,), jnp.float32),
            pltpu.VMEM((NQ,), jnp.int32),
            pltpu.VMEM((NQ,), jnp.float32),
        ],
    )
    return f(at, u2, sgn)


def kernel(assessment, maximize=False):
    key = jax.random.key(42)
    u = jax.random.uniform(key, (2, BATCH, NSEL))
    u2 = jnp.swapaxes(u, 0, 1).reshape(BATCH, NQ)
    at = jnp.swapaxes(assessment, 0, 1)
    maxi = jnp.asarray(maximize, jnp.bool_)
    sgn = jnp.where(maxi, jnp.float32(1), jnp.float32(-1)) * jnp.ones(
        (L,), jnp.float32)
    sel2, val2 = _sc_call(at, u2, sgn)
    sel = sel2.reshape(BATCH, 2, NSEL)
    val = val2.reshape(BATCH, 2, NSEL)
    return (jnp.swapaxes(val[:, 0, :], 0, 1),
            jnp.swapaxes(sel[:, 0, :], 0, 1),
            jnp.swapaxes(val[:, 1, :], 0, 1),
            jnp.swapaxes(sel[:, 1, :], 0, 1))
